# baseline (speedup 1.0000x reference)
import functools

import jax
import jax.numpy as jnp
from jax import lax
from jax.experimental import pallas as pl
from jax.experimental.pallas import tpu as pltpu
from jax.experimental.pallas import tpu_sc as plsc

_CHUNK = 512


def _sc_body(chunk, n_chunks, panes_per_w, num_cores, cache_hbm, update_hbm,
             pos_hbm, out_hbm, stage_v, upd_v, pos_v, in_sem, out_sem):
    c = lax.axis_index("c")
    s = lax.axis_index("s")
    wid = s * num_cores + c
    pane0 = wid * panes_per_w

    pltpu.sync_copy(pos_hbm, pos_v)
    pos_vec = pos_v[...]
    pos = pl.multiple_of(pos_vec[0], 8)
    upd_len = update_hbm.shape[1]

    for p in range(panes_per_w):
        pane = pane0 + p

        def chunk_step(i, _):
            r0 = i * chunk
            cp_in = pltpu.make_async_copy(
                cache_hbm.at[pane, pl.ds(r0, chunk), :], stage_v, in_sem
            )
            cp_in.start()
            cp_in.wait()
            cp_out = pltpu.make_async_copy(
                stage_v, out_hbm.at[pane, pl.ds(r0, chunk), :], out_sem
            )
            cp_out.start()
            cp_out.wait()
            return 0

        lax.fori_loop(0, n_chunks, chunk_step, 0)

        cp_u = pltpu.make_async_copy(update_hbm.at[pane], upd_v, in_sem)
        cp_u.start()
        cp_u.wait()
        cp_w = pltpu.make_async_copy(
            upd_v, out_hbm.at[pane, pl.ds(pos, upd_len), :], out_sem
        )
        cp_w.start()
        cp_w.wait()


def kernel(cache, update, pos):
    b, h, s, d = cache.shape
    upd_len = update.shape[-2]
    bh = b * h
    cache3 = cache.reshape(bh, s, d)
    update3 = update.reshape(bh, upd_len, d)
    pos_arr = jnp.full((16,), pos, jnp.int32)

    info = plsc.get_sparse_core_info()
    nw = info.num_cores * info.num_subcores
    panes_per_w = bh // nw
    n_chunks = s // _CHUNK

    mesh = plsc.VectorSubcoreMesh(
        core_axis_name="c", subcore_axis_name="s"
    )
    body = functools.partial(
        _sc_body, _CHUNK, n_chunks, panes_per_w, info.num_cores
    )
    out3 = pl.kernel(
        body,
        out_type=jax.ShapeDtypeStruct((bh, s, d), cache.dtype),
        mesh=mesh,
        scratch_types=[
            pltpu.VMEM((_CHUNK, d), jnp.float32),
            pltpu.VMEM((upd_len, d), jnp.float32),
            pltpu.VMEM((16,), jnp.int32),
            pltpu.SemaphoreType.DMA,
            pltpu.SemaphoreType.DMA,
        ],
    )(cache3, update3, pos_arr)
    return out3.reshape(b, h, s, d)


# speedup vs baseline: 1.0848x; 1.0848x over previous
import functools

import jax
import jax.numpy as jnp
from jax import lax
from jax.experimental import pallas as pl
from jax.experimental.pallas import tpu as pltpu
from jax.experimental.pallas import tpu_sc as plsc

_CHUNK = 256
_NBUF = 2


def _sc_body(n_chunks, rows_per_w, panes_per_w, num_cores, seq,
             cache_hbm, update_hbm, pos_hbm, out_hbm,
             stage_v, upd_v, pos_v, rd_sem, wr_sem, upd_sem):
    c = lax.axis_index("c")
    s = lax.axis_index("s")
    wid = s * num_cores + c
    row0 = wid * rows_per_w
    upd_len = upd_v.shape[1]

    pltpu.sync_copy(pos_hbm, pos_v)
    pos = pl.multiple_of(pos_v[...][0], 8)

    def rd(i, b):
        return pltpu.make_async_copy(
            cache_hbm.at[pl.ds(row0 + i * _CHUNK, _CHUNK), :],
            stage_v.at[b],
            rd_sem,
        )

    def wr(i, b):
        return pltpu.make_async_copy(
            stage_v.at[b],
            out_hbm.at[pl.ds(row0 + i * _CHUNK, _CHUNK), :],
            wr_sem,
        )

    for b in range(_NBUF):
        rd(b, b).start()

    def step(i, _):
        b = lax.rem(i, _NBUF)
        rd(i, b).wait()
        wr(i, b).start()

        @pl.when(i + _NBUF < n_chunks)
        def _refill():
            wr(i, b).wait()
            rd(i + _NBUF, b).start()

        return 0

    lax.fori_loop(0, n_chunks, step, 0)
    for k in range(_NBUF):
        wr(n_chunks - _NBUF + k, k).wait()

    for p in range(panes_per_w):
        pane = wid * panes_per_w + p
        cp_u = pltpu.make_async_copy(
            update_hbm.at[pl.ds(pane * upd_len, upd_len), :],
            upd_v.at[0],
            upd_sem,
        )
        cp_u.start()
        cp_u.wait()
        cp_w = pltpu.make_async_copy(
            upd_v.at[0],
            out_hbm.at[pl.ds(pane * seq + pos, upd_len), :],
            upd_sem,
        )
        cp_w.start()
        cp_w.wait()


def kernel(cache, update, pos):
    b, h, s, d = cache.shape
    upd_len = update.shape[-2]
    bh = b * h
    cache2 = cache.reshape(bh * s, d)
    update2 = update.reshape(bh * upd_len, d)
    pos_arr = jnp.full((16,), pos, jnp.int32)

    info = plsc.get_sparse_core_info()
    nw = info.num_cores * info.num_subcores
    panes_per_w = bh // nw
    rows_per_w = bh * s // nw
    n_chunks = rows_per_w // _CHUNK

    mesh = plsc.VectorSubcoreMesh(core_axis_name="c", subcore_axis_name="s")
    body = functools.partial(
        _sc_body, n_chunks, rows_per_w, panes_per_w, info.num_cores, s
    )
    out2 = pl.kernel(
        body,
        out_type=jax.ShapeDtypeStruct((bh * s, d), cache.dtype),
        mesh=mesh,
        scratch_types=[
            pltpu.VMEM((_NBUF, _CHUNK, d), jnp.float32),
            pltpu.VMEM((1, upd_len, d), jnp.float32),
            pltpu.VMEM((16,), jnp.int32),
            pltpu.SemaphoreType.DMA,
            pltpu.SemaphoreType.DMA,
            pltpu.SemaphoreType.DMA,
        ],
    )(cache2, update2, pos_arr)
    return out2.reshape(b, h, s, d)


# speedup vs baseline: 1.1963x; 1.1028x over previous
import functools

import jax
import jax.numpy as jnp
from jax import lax
from jax.experimental import pallas as pl
from jax.experimental.pallas import tpu as pltpu
from jax.experimental.pallas import tpu_sc as plsc

_SEQ_BLK = 4096
_BH_BLK = 4


def _copy_body(cache_ref, out_ref):
    out_ref[...] = cache_ref[...]


def _sc_scatter_body(panes_per_w, num_cores, seq,
                     update_hbm, pos_hbm, out_ref, upd_v, pos_v, sem):
    c = lax.axis_index("c")
    s = lax.axis_index("s")
    wid = s * num_cores + c
    upd_len = upd_v.shape[1]

    pltpu.sync_copy(pos_hbm, pos_v)
    pos = pl.multiple_of(pos_v[...][0], 8)

    for p in range(panes_per_w):
        pane = wid * panes_per_w + p
        cp_u = pltpu.make_async_copy(
            update_hbm.at[pl.ds(pane * upd_len, upd_len), :],
            upd_v.at[0],
            sem,
        )
        cp_u.start()
        cp_u.wait()
        cp_w = pltpu.make_async_copy(
            upd_v.at[0],
            out_ref.at[pl.ds(pane * seq + pos, upd_len), :],
            sem,
        )
        cp_w.start()
        cp_w.wait()


def kernel(cache, update, pos):
    b, h, s, d = cache.shape
    upd_len = update.shape[-2]
    bh = b * h
    cache3 = cache.reshape(bh, s, d)
    update2 = update.reshape(bh * upd_len, d)
    pos_arr = jnp.full((16,), pos, jnp.int32)

    cloned = pl.pallas_call(
        _copy_body,
        grid=(bh // _BH_BLK,),
        in_specs=[pl.BlockSpec((_BH_BLK, _SEQ_BLK, d), lambda i: (i, 0, 0))],
        out_specs=pl.BlockSpec((_BH_BLK, _SEQ_BLK, d), lambda i: (i, 0, 0)),
        out_shape=jax.ShapeDtypeStruct((bh, s, d), cache.dtype),
    )(cache3)

    info = plsc.get_sparse_core_info()
    nw = info.num_cores * info.num_subcores
    panes_per_w = bh // nw

    out_ref = jax.new_ref(cloned.reshape(bh * s, d))
    mesh = plsc.VectorSubcoreMesh(core_axis_name="c", subcore_axis_name="s")
    body = functools.partial(
        _sc_scatter_body, panes_per_w, info.num_cores, s
    )
    pl.kernel(
        body,
        out_type=(),
        mesh=mesh,
        scratch_types=[
            pltpu.VMEM((1, upd_len, d), jnp.float32),
            pltpu.VMEM((16,), jnp.int32),
            pltpu.SemaphoreType.DMA,
        ],
    )(update2, pos_arr, out_ref)
    return out_ref[...].reshape(b, h, s, d)


# speedup vs baseline: 1.3539x; 1.1317x over previous
import jax
import jax.numpy as jnp
from jax.experimental import pallas as pl
from jax.experimental.pallas import tpu as pltpu

_SEQ_BLK = 2048
_BH_BLK = 8


def _copy_body(pos_ref, cache_ref, update_ref, out_ref):
    j = pl.program_id(1)
    pos = pos_ref[0]
    row0 = j * _SEQ_BLK
    upd_len = update_ref.shape[1]
    d = cache_ref.shape[2]

    out_ref[...] = cache_ref[...]

    overlaps = (pos < row0 + _SEQ_BLK) & (row0 < pos + upd_len)

    @pl.when(overlaps)
    def _patch():
        win = jnp.clip(pos - row0, 0, _SEQ_BLK - upd_len)
        rel = (
            jax.lax.broadcasted_iota(jnp.int32, (upd_len, d), 0)
            + row0
            + win
            - pos
        )
        for b in range(_BH_BLK):
            val = out_ref[b, pl.ds(win, upd_len), :]
            for k in range(upd_len):
                val = jnp.where(rel == k, update_ref[b, k, :][None, :], val)
            out_ref[b, pl.ds(win, upd_len), :] = val


def kernel(cache, update, pos):
    b, h, s, d = cache.shape
    upd_len = update.shape[-2]
    cache3 = cache.reshape(b * h, s, d)
    update3 = update.reshape(b * h, upd_len, d)
    pos_arr = jnp.asarray(pos, jnp.int32).reshape((1,))

    grid = (b * h // _BH_BLK, s // _SEQ_BLK)
    out3 = pl.pallas_call(
        _copy_body,
        grid_spec=pltpu.PrefetchScalarGridSpec(
            num_scalar_prefetch=1,
            grid=grid,
            in_specs=[
                pl.BlockSpec(
                    (_BH_BLK, _SEQ_BLK, d), lambda i, j, pos_ref: (i, j, 0)
                ),
                pl.BlockSpec(
                    (_BH_BLK, upd_len, d), lambda i, j, pos_ref: (i, 0, 0)
                ),
            ],
            out_specs=pl.BlockSpec(
                (_BH_BLK, _SEQ_BLK, d), lambda i, j, pos_ref: (i, j, 0)
            ),
        ),
        out_shape=jax.ShapeDtypeStruct((b * h, s, d), cache.dtype),
        compiler_params=pltpu.CompilerParams(
            vmem_limit_bytes=100 * 1024 * 1024,
        ),
    )(pos_arr, cache3, update3)
    return out3.reshape(b, h, s, d)
